# Initial kernel scaffold; baseline (speedup 1.0000x reference)
import jax
import jax.numpy as jnp
from jax.experimental import pallas as pl


def kernel(users, pos_items, item_group_idx, user_embed, item_embed):
    raise NotImplementedError("write your pallas kernel here")


# speedup vs baseline: 1.6873x; 1.6873x over previous
import functools

import jax
import jax.numpy as jnp
from jax import lax
from jax.experimental import pallas as pl
from jax.experimental.pallas import tpu as pltpu
from jax.experimental.pallas import tpu_sc as plsc

N_ITEMS = 1_000_000
DIM = 64
N_GROUPS = 2000
BATCH = 16384
TAU = 0.1
DECAY = 1e-4

NC = 2
NS = 16
NW = NC * NS
CH = 64
NCH = N_ITEMS // CH
CH_ROUNDS = -(-NCH // NW)
ACC_W = DIM + 16
ZROWS = N_GROUPS // NS
GB = BATCH // NW
GQ = 128


def _rsqrt16(s):
    y = plsc.bitcast(
        jnp.int32(0x5F3759DF) - (plsc.bitcast(s, jnp.int32) >> 1), jnp.float32)
    half = s * 0.5
    for _ in range(4):
        y = y * (1.5 - half * y * y)
    return jnp.minimum(y, 1e12)


def _make_mesh():
    return plsc.VectorSubcoreMesh(core_axis_name="c", subcore_axis_name="s")


@functools.partial(
    pl.kernel,
    out_type=jax.ShapeDtypeStruct((NC, N_GROUPS, ACC_W), jnp.float32),
    mesh=_make_mesh(),
    compiler_params=pltpu.CompilerParams(use_tc_tiling_on_sc=False, needs_layout_passes=False),
    scratch_types=[
        pltpu.VMEM((CH * DIM,), jnp.float32),
        pltpu.VMEM((CH, ACC_W), jnp.float32),
        pltpu.VMEM((CH,), jnp.int32),
        pltpu.VMEM((CH,), jnp.float32),
        pltpu.VMEM((ZROWS, ACC_W), jnp.float32),
        pltpu.VMEM_SHARED((N_GROUPS, ACC_W), jnp.float32),
    ],
)
def _segment_kernel(iemb_flat, gidx, out, in_buf, norm_buf, idx_buf, rbuf,
                    zbuf, acc):
    c = lax.axis_index("c")
    s = lax.axis_index("s")
    w = s * NC + c
    zeros16 = jnp.zeros((16,), jnp.float32)
    ones16 = jnp.ones((16,), jnp.float32)

    def _zrow(i, carry):
        for q in range(ACC_W // 16):
            zbuf[i, pl.ds(q * 16, 16)] = zeros16
        return carry
    lax.fori_loop(0, ZROWS, _zrow, 0)

    def _onerow(i, carry):
        norm_buf[i, pl.ds(DIM, 16)] = ones16
        return carry
    lax.fori_loop(0, CH, _onerow, 0)

    pltpu.sync_copy(zbuf, acc.at[pl.ds(s * ZROWS, ZROWS)])
    plsc.subcore_barrier()

    rows0 = lax.iota(jnp.int32, 16)

    def _chunk(j, carry):
        k = w + NW * j

        @pl.when(k < NCH)
        def _():
            pltpu.sync_copy(iemb_flat.at[pl.ds(k * (CH * DIM), CH * DIM)],
                            in_buf)
            pltpu.sync_copy(gidx.at[pl.ds(k * CH, CH)], idx_buf)
            for j4 in range(CH // 16):
                addr = (rows0 + (j4 * 16)) * DIM
                ssq = zeros16
                for d in range(DIM):
                    v = plsc.load_gather(in_buf, [addr + d])
                    ssq = ssq + v * v
                rbuf[pl.ds(j4 * 16, 16)] = _rsqrt16(ssq)
            for r in range(CH):
                rsp = plsc.load_gather(rbuf, [jnp.full((16,), r, jnp.int32)])
                for q in range(DIM // 16):
                    v = in_buf[pl.ds(r * DIM + q * 16, 16)]
                    norm_buf[r, pl.ds(q * 16, 16)] = v * rsp
            pltpu.sync_copy(norm_buf, acc.at[idx_buf], add=True)
        return carry

    lax.fori_loop(0, CH_ROUNDS, _chunk, 0)
    plsc.subcore_barrier()

    @pl.when(s == 0)
    def _():
        pltpu.sync_copy(acc, out.at[c])


@functools.partial(
    pl.kernel,
    out_type=[
        jax.ShapeDtypeStruct((BATCH, DIM), jnp.float32),
        jax.ShapeDtypeStruct((BATCH, DIM), jnp.float32),
    ],
    mesh=_make_mesh(),
    compiler_params=pltpu.CompilerParams(use_tc_tiling_on_sc=False, needs_layout_passes=False),
    scratch_types=[
        pltpu.VMEM((GQ,), jnp.int32),
        pltpu.VMEM((GQ,), jnp.int32),
        pltpu.VMEM((GQ, DIM), jnp.float32),
        pltpu.VMEM((GQ, DIM), jnp.float32),
        pltpu.SemaphoreType.DMA,
        pltpu.SemaphoreType.DMA,
    ],
)
def _gather_kernel(users, pos, uemb, iemb, out_u, out_p,
                   uidx, pidx, ubuf, pbuf, sem_u, sem_p):
    c = lax.axis_index("c")
    s = lax.axis_index("s")
    w = s * NC + c
    base = w * GB

    def _q(q, carry):
        off = base + q * GQ
        pltpu.sync_copy(users.at[pl.ds(off, GQ)], uidx)
        pltpu.sync_copy(pos.at[pl.ds(off, GQ)], pidx)
        cp_u = pltpu.async_copy(uemb.at[uidx], ubuf, sem_u)
        cp_p = pltpu.async_copy(iemb.at[pidx], pbuf, sem_p)
        cp_u.wait()
        cp_p.wait()
        pltpu.sync_copy(ubuf, out_u.at[pl.ds(off, GQ)])
        pltpu.sync_copy(pbuf, out_p.at[pl.ds(off, GQ)])
        return carry

    lax.fori_loop(0, GB // GQ, _q, 0)


BB = 512
NB = BATCH // BB


def _loss_body(acc_ref, u_ref, p_ref, out_loss, out_emb, nege_s, smem):
    i = pl.program_id(0)

    @pl.when(i == 0)
    def _():
        sums = acc_ref[0, :, :DIM] + acc_ref[1, :, :DIM]
        cnt = acc_ref[0, :, DIM:DIM + 1] + acc_ref[1, :, DIM:DIM + 1]
        neg = sums / jnp.maximum(cnt, 1.0)
        smem[0] = jnp.sum(neg * neg)
        nrm = jnp.sqrt(jnp.sum(neg * neg, axis=-1, keepdims=True))
        nege_s[...] = neg / jnp.maximum(nrm, 1e-12)
        smem[1] = 0.0
        smem[2] = 0.0
        smem[3] = 0.0

    u = u_ref[...]
    p = p_ref[...]
    un = jnp.sqrt(jnp.sum(u * u, axis=-1, keepdims=True))
    ue = u / jnp.maximum(un, 1e-12)
    pn = jnp.sqrt(jnp.sum(p * p, axis=-1, keepdims=True))
    pe = p / jnp.maximum(pn, 1e-12)
    ypos = jnp.sum(ue * pe, axis=-1)
    y = lax.dot_general(ue, nege_s[...], (((1,), (1,)), ((), ())),
                        preferred_element_type=jnp.float32)
    z = (y - ypos[:, None]) * (1.0 / TAU)
    sp = jnp.maximum(z, 0.0) + jnp.log(1.0 + jnp.exp(-jnp.abs(z)))
    smem[1] = smem[1] + jnp.sum(sp)
    smem[2] = smem[2] + jnp.sum(u * u)
    smem[3] = smem[3] + jnp.sum(p * p)

    @pl.when(i == NB - 1)
    def _():
        reg = (smem[2] + smem[3] + smem[0]) * 0.5
        emb = DECAY * reg / BATCH
        out_emb[...] = jnp.broadcast_to(emb, (1, 1))
        out_loss[...] = jnp.broadcast_to(
            smem[1] / (BATCH * N_GROUPS) + emb, (1, 1))


def _loss_call(acc, u_raw, p_raw):
    return pl.pallas_call(
        _loss_body,
        grid=(NB,),
        in_specs=[
            pl.BlockSpec((NC, N_GROUPS, ACC_W), lambda i: (0, 0, 0)),
            pl.BlockSpec((BB, DIM), lambda i: (i, 0)),
            pl.BlockSpec((BB, DIM), lambda i: (i, 0)),
        ],
        out_specs=[
            pl.BlockSpec((1, 1), lambda i: (0, 0)),
            pl.BlockSpec((1, 1), lambda i: (0, 0)),
        ],
        out_shape=[
            jax.ShapeDtypeStruct((1, 1), jnp.float32),
            jax.ShapeDtypeStruct((1, 1), jnp.float32),
        ],
        scratch_shapes=[
            pltpu.VMEM((N_GROUPS, DIM), jnp.float32),
            pltpu.SMEM((4,), jnp.float32),
        ],
    )(acc, u_raw, p_raw)


def kernel(users, pos_items, item_group_idx, user_embed, item_embed):
    acc = _segment_kernel(item_embed.reshape(-1), item_group_idx)
    u_raw, p_raw = _gather_kernel(users, pos_items, user_embed, item_embed)
    loss, emb = _loss_call(acc, u_raw, p_raw)
    return loss[0, 0], emb[0, 0]


# speedup vs baseline: 1.9626x; 1.1632x over previous
import functools

import jax
import jax.numpy as jnp
from jax import lax
from jax.experimental import pallas as pl
from jax.experimental.pallas import tpu as pltpu
from jax.experimental.pallas import tpu_sc as plsc

N_ITEMS = 1_000_000
DIM = 64
N_GROUPS = 2000
BATCH = 16384
TAU = 0.1
DECAY = 1e-4

NC = 2
NS = 16
NW = NC * NS
CH = 64
NCH = N_ITEMS // CH
CH_ROUNDS = -(-NCH // NW)
ACC_W = DIM + 16
ZROWS = N_GROUPS // NS
GB = BATCH // NW
GQ = 128


def _rsqrt16(s):
    y = plsc.bitcast(
        jnp.int32(0x5F3759DF) - (plsc.bitcast(s, jnp.int32) >> 1), jnp.float32)
    half = s * 0.5
    for _ in range(4):
        y = y * (1.5 - half * y * y)
    return jnp.minimum(y, 1e12)


def _make_mesh():
    return plsc.VectorSubcoreMesh(core_axis_name="c", subcore_axis_name="s")


@functools.partial(
    pl.kernel,
    out_type=jax.ShapeDtypeStruct((NC, N_GROUPS, ACC_W), jnp.float32),
    mesh=_make_mesh(),
    compiler_params=pltpu.CompilerParams(use_tc_tiling_on_sc=False, needs_layout_passes=False),
    scratch_types=[
        pltpu.VMEM((CH * DIM,), jnp.float32),
        pltpu.VMEM((CH * DIM,), jnp.float32),
        pltpu.VMEM((CH, ACC_W), jnp.float32),
        pltpu.VMEM((CH,), jnp.int32),
        pltpu.VMEM((CH,), jnp.int32),
        pltpu.VMEM((CH,), jnp.float32),
        pltpu.VMEM((ZROWS, ACC_W), jnp.float32),
        pltpu.VMEM_SHARED((N_GROUPS, ACC_W), jnp.float32),
        pltpu.SemaphoreType.DMA,
        pltpu.SemaphoreType.DMA,
    ],
)
def _segment_kernel(iemb_flat, gidx, out, in0, in1, norm_buf, ix0, ix1, rbuf,
                    zbuf, acc, sem0, sem1):
    c = lax.axis_index("c")
    s = lax.axis_index("s")
    w = s * NC + c
    zeros16 = jnp.zeros((16,), jnp.float32)
    ones16 = jnp.ones((16,), jnp.float32)
    ins = (in0, in1)
    ixs = (ix0, ix1)
    sems = (sem0, sem1)

    def _zrow(i, carry):
        for q in range(ACC_W // 16):
            zbuf[i, pl.ds(q * 16, 16)] = zeros16
        return carry
    lax.fori_loop(0, ZROWS, _zrow, 0)

    def _onerow(i, carry):
        norm_buf[i, pl.ds(DIM, 16)] = ones16
        return carry
    lax.fori_loop(0, CH, _onerow, 0)

    pltpu.sync_copy(zbuf, acc.at[pl.ds(s * ZROWS, ZROWS)])
    plsc.subcore_barrier()

    rows0 = lax.iota(jnp.int32, 16)

    def _issue_rows(slot, k):
        pltpu.async_copy(iemb_flat.at[pl.ds(k * (CH * DIM), CH * DIM)],
                         ins[slot], sems[slot])

    def _issue_idx(slot, k):
        pltpu.async_copy(gidx.at[pl.ds(k * CH, CH)], ixs[slot], sems[slot])

    def _wait_in(slot, k):
        pltpu.make_async_copy(iemb_flat.at[pl.ds(k * (CH * DIM), CH * DIM)],
                              ins[slot], sems[slot]).wait()
        pltpu.make_async_copy(gidx.at[pl.ds(k * CH, CH)], ixs[slot],
                              sems[slot]).wait()

    _issue_rows(0, w)
    _issue_idx(0, w)
    _issue_rows(1, w + NW)
    _issue_idx(1, w + NW)

    def _round(slot, j):
        k = w + NW * j

        @pl.when(k < NCH)
        def _():
            _wait_in(slot, k)
            in_buf = ins[slot]
            for j4 in range(CH // 16):
                addr = (rows0 + (j4 * 16)) * DIM
                ssq = zeros16
                for d in range(DIM):
                    v = plsc.load_gather(in_buf, [addr + d])
                    ssq = ssq + v * v
                rbuf[pl.ds(j4 * 16, 16)] = _rsqrt16(ssq)
            for r in range(CH):
                rsp = plsc.load_gather(rbuf, [jnp.full((16,), r, jnp.int32)])
                for q in range(DIM // 16):
                    v = in_buf[pl.ds(r * DIM + q * 16, 16)]
                    norm_buf[r, pl.ds(q * 16, 16)] = v * rsp
            @pl.when(k + 2 * NW < NCH)
            def _():
                _issue_rows(slot, k + 2 * NW)
            pltpu.sync_copy(norm_buf, acc.at[ixs[slot]], add=True)

            @pl.when(k + 2 * NW < NCH)
            def _():
                _issue_idx(slot, k + 2 * NW)

    def _pair(p, carry):
        _round(0, 2 * p)
        _round(1, 2 * p + 1)
        return carry

    lax.fori_loop(0, (CH_ROUNDS + 1) // 2, _pair, 0)
    plsc.subcore_barrier()

    @pl.when(s == 0)
    def _():
        pltpu.sync_copy(acc, out.at[c])


@functools.partial(
    pl.kernel,
    out_type=[
        jax.ShapeDtypeStruct((BATCH, DIM), jnp.float32),
        jax.ShapeDtypeStruct((BATCH, DIM), jnp.float32),
    ],
    mesh=_make_mesh(),
    compiler_params=pltpu.CompilerParams(use_tc_tiling_on_sc=False, needs_layout_passes=False),
    scratch_types=[
        pltpu.VMEM((GQ,), jnp.int32),
        pltpu.VMEM((GQ,), jnp.int32),
        pltpu.VMEM((GQ, DIM), jnp.float32),
        pltpu.VMEM((GQ, DIM), jnp.float32),
        pltpu.SemaphoreType.DMA,
        pltpu.SemaphoreType.DMA,
    ],
)
def _gather_kernel(users, pos, uemb, iemb, out_u, out_p,
                   uidx, pidx, ubuf, pbuf, sem_u, sem_p):
    c = lax.axis_index("c")
    s = lax.axis_index("s")
    w = s * NC + c
    base = w * GB

    def _q(q, carry):
        off = base + q * GQ
        pltpu.sync_copy(users.at[pl.ds(off, GQ)], uidx)
        pltpu.sync_copy(pos.at[pl.ds(off, GQ)], pidx)
        cp_u = pltpu.async_copy(uemb.at[uidx], ubuf, sem_u)
        cp_p = pltpu.async_copy(iemb.at[pidx], pbuf, sem_p)
        cp_u.wait()
        cp_p.wait()
        pltpu.sync_copy(ubuf, out_u.at[pl.ds(off, GQ)])
        pltpu.sync_copy(pbuf, out_p.at[pl.ds(off, GQ)])
        return carry

    lax.fori_loop(0, GB // GQ, _q, 0)


BB = 512
NB = BATCH // BB


def _loss_body(acc_ref, u_ref, p_ref, out_loss, out_emb, nege_s, smem):
    i = pl.program_id(0)

    @pl.when(i == 0)
    def _():
        sums = acc_ref[0, :, :DIM] + acc_ref[1, :, :DIM]
        cnt = acc_ref[0, :, DIM:DIM + 1] + acc_ref[1, :, DIM:DIM + 1]
        neg = sums / jnp.maximum(cnt, 1.0)
        smem[0] = jnp.sum(neg * neg)
        nrm = jnp.sqrt(jnp.sum(neg * neg, axis=-1, keepdims=True))
        nege_s[...] = neg / jnp.maximum(nrm, 1e-12)
        smem[1] = 0.0
        smem[2] = 0.0
        smem[3] = 0.0

    u = u_ref[...]
    p = p_ref[...]
    un = jnp.sqrt(jnp.sum(u * u, axis=-1, keepdims=True))
    ue = u / jnp.maximum(un, 1e-12)
    pn = jnp.sqrt(jnp.sum(p * p, axis=-1, keepdims=True))
    pe = p / jnp.maximum(pn, 1e-12)
    ypos = jnp.sum(ue * pe, axis=-1)
    y = lax.dot_general(ue, nege_s[...], (((1,), (1,)), ((), ())),
                        preferred_element_type=jnp.float32)
    z = (y - ypos[:, None]) * (1.0 / TAU)
    sp = jnp.maximum(z, 0.0) + jnp.log(1.0 + jnp.exp(-jnp.abs(z)))
    smem[1] = smem[1] + jnp.sum(sp)
    smem[2] = smem[2] + jnp.sum(u * u)
    smem[3] = smem[3] + jnp.sum(p * p)

    @pl.when(i == NB - 1)
    def _():
        reg = (smem[2] + smem[3] + smem[0]) * 0.5
        emb = DECAY * reg / BATCH
        out_emb[...] = jnp.broadcast_to(emb, (1, 1))
        out_loss[...] = jnp.broadcast_to(
            smem[1] / (BATCH * N_GROUPS) + emb, (1, 1))


def _loss_call(acc, u_raw, p_raw):
    return pl.pallas_call(
        _loss_body,
        grid=(NB,),
        in_specs=[
            pl.BlockSpec((NC, N_GROUPS, ACC_W), lambda i: (0, 0, 0)),
            pl.BlockSpec((BB, DIM), lambda i: (i, 0)),
            pl.BlockSpec((BB, DIM), lambda i: (i, 0)),
        ],
        out_specs=[
            pl.BlockSpec((1, 1), lambda i: (0, 0)),
            pl.BlockSpec((1, 1), lambda i: (0, 0)),
        ],
        out_shape=[
            jax.ShapeDtypeStruct((1, 1), jnp.float32),
            jax.ShapeDtypeStruct((1, 1), jnp.float32),
        ],
        scratch_shapes=[
            pltpu.VMEM((N_GROUPS, DIM), jnp.float32),
            pltpu.SMEM((4,), jnp.float32),
        ],
    )(acc, u_raw, p_raw)


def kernel(users, pos_items, item_group_idx, user_embed, item_embed):
    acc = _segment_kernel(item_embed.reshape(-1), item_group_idx)
    u_raw, p_raw = _gather_kernel(users, pos_items, user_embed, item_embed)
    loss, emb = _loss_call(acc, u_raw, p_raw)
    return loss[0, 0], emb[0, 0]


# speedup vs baseline: 2.0235x; 1.0310x over previous
import functools

import jax
import jax.numpy as jnp
from jax import lax
from jax.experimental import pallas as pl
from jax.experimental.pallas import tpu as pltpu
from jax.experimental.pallas import tpu_sc as plsc

N_ITEMS = 1_000_000
DIM = 64
N_GROUPS = 2000
BATCH = 16384
TAU = 0.1
DECAY = 1e-4

NC = 2
NS = 16
NW = NC * NS
CH = 64
NCH = N_ITEMS // CH
CH_ROUNDS = -(-NCH // NW)
ACC_W = DIM + 16
ZROWS = N_GROUPS // NS
GB = BATCH // NW
GQ = 128


def _rsqrt16(s):
    y = plsc.bitcast(
        jnp.int32(0x5F3759DF) - (plsc.bitcast(s, jnp.int32) >> 1), jnp.float32)
    half = s * 0.5
    for _ in range(4):
        y = y * (1.5 - half * y * y)
    return jnp.minimum(y, 1e12)


def _make_mesh():
    return plsc.VectorSubcoreMesh(core_axis_name="c", subcore_axis_name="s")


@functools.partial(
    pl.kernel,
    out_type=jax.ShapeDtypeStruct((NC, N_GROUPS, ACC_W), jnp.float32),
    mesh=_make_mesh(),
    compiler_params=pltpu.CompilerParams(use_tc_tiling_on_sc=False, needs_layout_passes=False),
    scratch_types=[
        pltpu.VMEM((CH * DIM,), jnp.float32),
        pltpu.VMEM((CH * DIM,), jnp.float32),
        pltpu.VMEM((CH, ACC_W), jnp.float32),
        pltpu.VMEM((CH,), jnp.int32),
        pltpu.VMEM((CH,), jnp.int32),
        pltpu.VMEM((CH,), jnp.float32),
        pltpu.VMEM((ZROWS, ACC_W), jnp.float32),
        pltpu.VMEM_SHARED((N_GROUPS, ACC_W), jnp.float32),
        pltpu.SemaphoreType.DMA,
        pltpu.SemaphoreType.DMA,
    ],
)
def _segment_kernel(iemb_flat, gidx, out, in0, in1, norm_buf, ix0, ix1, rbuf,
                    zbuf, acc, sem0, sem1):
    c = lax.axis_index("c")
    s = lax.axis_index("s")
    w = s * NC + c
    zeros16 = jnp.zeros((16,), jnp.float32)
    ones16 = jnp.ones((16,), jnp.float32)
    ins = (in0, in1)
    ixs = (ix0, ix1)
    sems = (sem0, sem1)

    def _zrow(i, carry):
        for q in range(ACC_W // 16):
            zbuf[i, pl.ds(q * 16, 16)] = zeros16
        return carry
    lax.fori_loop(0, ZROWS, _zrow, 0)

    def _onerow(i, carry):
        norm_buf[i, pl.ds(DIM, 16)] = ones16
        return carry
    lax.fori_loop(0, CH, _onerow, 0)

    pltpu.sync_copy(zbuf, acc.at[pl.ds(s * ZROWS, ZROWS)])
    plsc.subcore_barrier()

    rows0 = lax.iota(jnp.int32, 16)

    def _issue_rows(slot, k):
        pltpu.async_copy(iemb_flat.at[pl.ds(k * (CH * DIM), CH * DIM)],
                         ins[slot], sems[slot])

    def _issue_idx(slot, k):
        pltpu.async_copy(gidx.at[pl.ds(k * CH, CH)], ixs[slot], sems[slot])

    def _wait_in(slot, k):
        pltpu.make_async_copy(iemb_flat.at[pl.ds(k * (CH * DIM), CH * DIM)],
                              ins[slot], sems[slot]).wait()
        pltpu.make_async_copy(gidx.at[pl.ds(k * CH, CH)], ixs[slot],
                              sems[slot]).wait()

    _issue_rows(0, w)
    _issue_idx(0, w)
    _issue_rows(1, w + NW)
    _issue_idx(1, w + NW)

    def _round(slot, j):
        k = w + NW * j

        @pl.when(k < NCH)
        def _():
            _wait_in(slot, k)
            in_buf = ins[slot]
            for j4 in range(CH // 16):
                rows = rows0 + (j4 * 16)
                rowbase = rows * DIM
                ssq = zeros16
                for t in range(DIM):
                    col = (rows0 + t) & (DIM - 1)
                    v = plsc.load_gather(in_buf, [rowbase + col])
                    ssq = ssq + v * v
                r = _rsqrt16(ssq)
                for t in range(DIM):
                    col = (rows0 + t) & (DIM - 1)
                    v = plsc.load_gather(in_buf, [rowbase + col])
                    plsc.store_scatter(norm_buf, [rows, col], v * r)
            @pl.when(k + 2 * NW < NCH)
            def _():
                _issue_rows(slot, k + 2 * NW)
            pltpu.sync_copy(norm_buf, acc.at[ixs[slot]], add=True)

            @pl.when(k + 2 * NW < NCH)
            def _():
                _issue_idx(slot, k + 2 * NW)

    def _pair(p, carry):
        _round(0, 2 * p)
        _round(1, 2 * p + 1)
        return carry

    lax.fori_loop(0, (CH_ROUNDS + 1) // 2, _pair, 0)
    plsc.subcore_barrier()

    @pl.when(s == 0)
    def _():
        pltpu.sync_copy(acc, out.at[c])


@functools.partial(
    pl.kernel,
    out_type=[
        jax.ShapeDtypeStruct((BATCH, DIM), jnp.float32),
        jax.ShapeDtypeStruct((BATCH, DIM), jnp.float32),
    ],
    mesh=_make_mesh(),
    compiler_params=pltpu.CompilerParams(use_tc_tiling_on_sc=False, needs_layout_passes=False),
    scratch_types=[
        pltpu.VMEM((GQ,), jnp.int32),
        pltpu.VMEM((GQ,), jnp.int32),
        pltpu.VMEM((GQ, DIM), jnp.float32),
        pltpu.VMEM((GQ, DIM), jnp.float32),
        pltpu.SemaphoreType.DMA,
        pltpu.SemaphoreType.DMA,
    ],
)
def _gather_kernel(users, pos, uemb, iemb, out_u, out_p,
                   uidx, pidx, ubuf, pbuf, sem_u, sem_p):
    c = lax.axis_index("c")
    s = lax.axis_index("s")
    w = s * NC + c
    base = w * GB

    def _q(q, carry):
        off = base + q * GQ
        pltpu.sync_copy(users.at[pl.ds(off, GQ)], uidx)
        pltpu.sync_copy(pos.at[pl.ds(off, GQ)], pidx)
        cp_u = pltpu.async_copy(uemb.at[uidx], ubuf, sem_u)
        cp_p = pltpu.async_copy(iemb.at[pidx], pbuf, sem_p)
        cp_u.wait()
        cp_p.wait()
        pltpu.sync_copy(ubuf, out_u.at[pl.ds(off, GQ)])
        pltpu.sync_copy(pbuf, out_p.at[pl.ds(off, GQ)])
        return carry

    lax.fori_loop(0, GB // GQ, _q, 0)


BB = 512
NB = BATCH // BB


def _loss_body(acc_ref, u_ref, p_ref, out_loss, out_emb, nege_s, smem):
    i = pl.program_id(0)

    @pl.when(i == 0)
    def _():
        sums = acc_ref[0, :, :DIM] + acc_ref[1, :, :DIM]
        cnt = acc_ref[0, :, DIM:DIM + 1] + acc_ref[1, :, DIM:DIM + 1]
        neg = sums / jnp.maximum(cnt, 1.0)
        smem[0] = jnp.sum(neg * neg)
        nrm = jnp.sqrt(jnp.sum(neg * neg, axis=-1, keepdims=True))
        nege_s[...] = neg / jnp.maximum(nrm, 1e-12)
        smem[1] = 0.0
        smem[2] = 0.0
        smem[3] = 0.0

    u = u_ref[...]
    p = p_ref[...]
    un = jnp.sqrt(jnp.sum(u * u, axis=-1, keepdims=True))
    ue = u / jnp.maximum(un, 1e-12)
    pn = jnp.sqrt(jnp.sum(p * p, axis=-1, keepdims=True))
    pe = p / jnp.maximum(pn, 1e-12)
    ypos = jnp.sum(ue * pe, axis=-1)
    y = lax.dot_general(ue, nege_s[...], (((1,), (1,)), ((), ())),
                        preferred_element_type=jnp.float32)
    z = (y - ypos[:, None]) * (1.0 / TAU)
    sp = jnp.maximum(z, 0.0) + jnp.log(1.0 + jnp.exp(-jnp.abs(z)))
    smem[1] = smem[1] + jnp.sum(sp)
    smem[2] = smem[2] + jnp.sum(u * u)
    smem[3] = smem[3] + jnp.sum(p * p)

    @pl.when(i == NB - 1)
    def _():
        reg = (smem[2] + smem[3] + smem[0]) * 0.5
        emb = DECAY * reg / BATCH
        out_emb[...] = jnp.broadcast_to(emb, (1, 1))
        out_loss[...] = jnp.broadcast_to(
            smem[1] / (BATCH * N_GROUPS) + emb, (1, 1))


def _loss_call(acc, u_raw, p_raw):
    return pl.pallas_call(
        _loss_body,
        grid=(NB,),
        in_specs=[
            pl.BlockSpec((NC, N_GROUPS, ACC_W), lambda i: (0, 0, 0)),
            pl.BlockSpec((BB, DIM), lambda i: (i, 0)),
            pl.BlockSpec((BB, DIM), lambda i: (i, 0)),
        ],
        out_specs=[
            pl.BlockSpec((1, 1), lambda i: (0, 0)),
            pl.BlockSpec((1, 1), lambda i: (0, 0)),
        ],
        out_shape=[
            jax.ShapeDtypeStruct((1, 1), jnp.float32),
            jax.ShapeDtypeStruct((1, 1), jnp.float32),
        ],
        scratch_shapes=[
            pltpu.VMEM((N_GROUPS, DIM), jnp.float32),
            pltpu.SMEM((4,), jnp.float32),
        ],
    )(acc, u_raw, p_raw)


def kernel(users, pos_items, item_group_idx, user_embed, item_embed):
    acc = _segment_kernel(item_embed.reshape(-1), item_group_idx)
    u_raw, p_raw = _gather_kernel(users, pos_items, user_embed, item_embed)
    loss, emb = _loss_call(acc, u_raw, p_raw)
    return loss[0, 0], emb[0, 0]
